# Initial kernel scaffold; baseline (speedup 1.0000x reference)
import functools

import jax
import jax.numpy as jnp
from jax import lax
from jax.experimental import pallas as pl
from jax.experimental.pallas import tpu as pltpu
from jax.experimental.pallas import tpu_sc as plsc

_NC = 2
_NS = 16
_NW = _NC * _NS
_CH = 128


def _make_gather(V, D, B):
  b_per_w = B // _NW
  n_chunks = b_per_w // _CH
  mesh = plsc.VectorSubcoreMesh(core_axis_name="c", subcore_axis_name="s")

  @functools.partial(
      pl.kernel,
      mesh=mesh,
      out_type=jax.ShapeDtypeStruct((B, D), jnp.float32),
      scratch_types=[
          pltpu.VMEM((n_chunks, _CH), jnp.int32),
          pltpu.VMEM((_CH, D), jnp.float32),
          pltpu.SemaphoreType.DMA,
      ],
  )
  def emb(table_hbm, idx_hbm, out_hbm, idx_v, rows_v, sem):
    wid = lax.axis_index("s") * _NC + lax.axis_index("c")
    base = wid * b_per_w
    pltpu.sync_copy(idx_hbm.at[wid], idx_v)

    def body(j, carry):
      pltpu.async_copy(table_hbm.at[idx_v.at[j]], rows_v, sem).wait()
      pltpu.sync_copy(rows_v, out_hbm.at[pl.ds(base + j * _CH, _CH)])
      return carry

    lax.fori_loop(0, n_chunks, body, 0)

  return emb


def kernel(input_ids, use_one_hot_embeddings, embedding_table):
  V, D = embedding_table.shape
  orig_shape = input_ids.shape
  flat = input_ids.reshape(-1)
  B = flat.shape[0]
  b_per_w = B // _NW
  n_chunks = b_per_w // _CH
  idx3 = flat.reshape(_NW, n_chunks, _CH)

  out = _make_gather(V, D, B)(embedding_table, idx3)
  out = out.reshape(orig_shape + (D,))
  return (out, embedding_table)


# speedup vs baseline: 1.0742x; 1.0742x over previous
import functools

import jax
import jax.numpy as jnp
from jax import lax
from jax.experimental import pallas as pl
from jax.experimental.pallas import tpu as pltpu
from jax.experimental.pallas import tpu_sc as plsc

_NC = 2
_NS = 16
_NW = _NC * _NS
_CH = 128


def _make_gather(V, D, B):
  b_per_w = B // _NW
  n_chunks = b_per_w // _CH
  mesh = plsc.VectorSubcoreMesh(core_axis_name="c", subcore_axis_name="s")

  @functools.partial(
      pl.kernel,
      mesh=mesh,
      compiler_params=pltpu.CompilerParams(use_tc_tiling_on_sc=False),
      out_type=jax.ShapeDtypeStruct((B, D), jnp.float32),
      scratch_types=[
          pltpu.VMEM((n_chunks, _CH), jnp.int32),
          pltpu.VMEM((_CH, D), jnp.float32),
          pltpu.SemaphoreType.DMA,
      ],
  )
  def emb(table_hbm, idx_hbm, out_hbm, idx_v, rows_v, sem):
    wid = lax.axis_index("s") * _NC + lax.axis_index("c")
    base = wid * b_per_w
    pltpu.sync_copy(idx_hbm.at[wid], idx_v)

    def body(j, carry):
      pltpu.async_copy(table_hbm.at[idx_v.at[j]], rows_v, sem).wait()
      pltpu.sync_copy(rows_v, out_hbm.at[pl.ds(base + j * _CH, _CH)])
      return carry

    lax.fori_loop(0, n_chunks, body, 0)

  return emb


def kernel(input_ids, use_one_hot_embeddings, embedding_table):
  V, D = embedding_table.shape
  orig_shape = input_ids.shape
  flat = input_ids.reshape(-1)
  B = flat.shape[0]
  b_per_w = B // _NW
  n_chunks = b_per_w // _CH
  idx3 = flat.reshape(_NW, n_chunks, _CH)

  out = _make_gather(V, D, B)(embedding_table, idx3)
  out = out.reshape(orig_shape + (D,))
  return (out, embedding_table)


# speedup vs baseline: 1.1145x; 1.0374x over previous
import functools

import jax
import jax.numpy as jnp
from jax import lax
from jax.experimental import pallas as pl
from jax.experimental.pallas import tpu as pltpu
from jax.experimental.pallas import tpu_sc as plsc

_NC = 2
_NS = 16
_NW = _NC * _NS
_CH = 128
_K = 5


def _make_gather(V, D, B):
  b_per_w = B // _NW
  n_chunks = b_per_w // _CH
  n_groups = n_chunks // _K
  n_pairs = n_groups // 2
  grp_rows = _K * _CH
  mesh = plsc.VectorSubcoreMesh(core_axis_name="c", subcore_axis_name="s")

  @functools.partial(
      pl.kernel,
      mesh=mesh,
      compiler_params=pltpu.CompilerParams(use_tc_tiling_on_sc=False),
      out_type=jax.ShapeDtypeStruct((B, D), jnp.float32),
      scratch_types=[
          pltpu.VMEM((n_chunks, _CH), jnp.int32),
          pltpu.VMEM((grp_rows, D), jnp.float32),
          pltpu.VMEM((grp_rows, D), jnp.float32),
          pltpu.SemaphoreType.DMA,
          pltpu.SemaphoreType.DMA,
          pltpu.SemaphoreType.DMA,
          pltpu.SemaphoreType.DMA,
      ],
  )
  def emb(table_hbm, idx_hbm, out_hbm, idx_v, rows_a, rows_b,
          gsem_a, gsem_b, osem_a, osem_b):
    wid = lax.axis_index("s") * _NC + lax.axis_index("c")
    base = wid * b_per_w
    pltpu.sync_copy(idx_hbm.at[wid], idx_v)

    def fire_gathers(g, rows, gsem):
      for b in range(_K):
        pltpu.async_copy(
            table_hbm.at[idx_v.at[g * _K + b]],
            rows.at[pl.ds(b * _CH, _CH)], gsem)

    def drain_gathers(g, rows, gsem):
      for b in range(_K):
        pltpu.make_async_copy(
            table_hbm.at[idx_v.at[g * _K + b]],
            rows.at[pl.ds(b * _CH, _CH)], gsem).wait()

    def fire_wb(g, rows, osem):
      pltpu.async_copy(rows, out_hbm.at[pl.ds(base + g * grp_rows, grp_rows)],
                       osem)

    def drain_wb(g, rows, osem):
      pltpu.make_async_copy(rows,
                            out_hbm.at[pl.ds(base + g * grp_rows, grp_rows)],
                            osem).wait()

    fire_gathers(0, rows_a, gsem_a)

    def body(p, carry):
      g0 = 2 * p
      g1 = g0 + 1

      @pl.when(p > 0)
      def _():
        drain_wb(g1, rows_b, osem_b)

      fire_gathers(g1, rows_b, gsem_b)
      drain_gathers(g0, rows_a, gsem_a)
      fire_wb(g0, rows_a, osem_a)

      @pl.when(p + 1 < n_pairs)
      def _():
        drain_wb(g0, rows_a, osem_a)
        fire_gathers(g0 + 2, rows_a, gsem_a)

      drain_gathers(g1, rows_b, gsem_b)
      fire_wb(g1, rows_b, osem_b)
      return carry

    lax.fori_loop(0, n_pairs, body, 0)
    drain_wb(0, rows_a, osem_a)
    drain_wb(0, rows_b, osem_b)

  return emb


def kernel(input_ids, use_one_hot_embeddings, embedding_table):
  V, D = embedding_table.shape
  orig_shape = input_ids.shape
  flat = input_ids.reshape(-1)
  B = flat.shape[0]
  b_per_w = B // _NW
  n_chunks = b_per_w // _CH
  idx3 = flat.reshape(_NW, n_chunks, _CH)

  out = _make_gather(V, D, B)(embedding_table, idx3)
  out = out.reshape(orig_shape + (D,))
  return (out, embedding_table)
